# Initial kernel scaffold; baseline (speedup 1.0000x reference)
import jax
import jax.numpy as jnp
from jax.experimental import pallas as pl


def kernel(node_feat, coord_feat, edge_feat, We1, be1, We2, be2, Wn1, bn1, Wn2, bn2, Wc1, bc1, Wc2, edge_index):
    raise NotImplementedError("write your pallas kernel here")


# speedup vs baseline: 3.2498x; 3.2498x over previous
import jax
import jax.numpy as jnp
from jax import lax
from jax.experimental import pallas as pl
from jax.experimental.pallas import tpu as pltpu
from jax.experimental.pallas import tpu_sc as plsc

F32 = jnp.float32
CHUNK = 128
NCORES = 2
NSUB = 16
NW = NCORES * NSUB
XW = 16


_DEBUG_XLA_SCATTER = False
_DEBUG_XLA_GATHER = False
_DEBUG_XLA_XA = False
_DEBUG_SC_XA = False


def _silu(x):
    return x * jax.nn.sigmoid(x)


def _sc_gather_body(nf_hbm, cx_hbm, cy_hbm, cz_hbm, src_hbm, dst_hbm,
                    hs_hbm, hd_hbm, xd_hbm,
                    idx_s, idx_d, hbuf_s, hbuf_d, cbuf,
                    ctx, cty, ctz, sem):
    c = lax.axis_index("c")
    s = lax.axis_index("s")
    wid = c * NSUB + s
    ep = hs_hbm.shape[0]
    epw = ep // NW
    nchunks = epw // CHUNK
    base0 = wid * epw

    pltpu.sync_copy(cx_hbm, ctx)
    pltpu.sync_copy(cy_hbm, cty)
    pltpu.sync_copy(cz_hbm, ctz)
    zero16 = jnp.zeros((16,), F32)
    for r in range(4, XW):
        for g in range(CHUNK // 16):
            cbuf[r, pl.ds(g * 16, 16)] = zero16

    def body(k, carry):
        base = base0 + k * CHUNK
        pltpu.sync_copy(src_hbm.at[pl.ds(base, CHUNK)], idx_s)
        pltpu.sync_copy(dst_hbm.at[pl.ds(base, CHUNK)], idx_d)
        g0 = pltpu.async_copy(nf_hbm.at[idx_s], hbuf_s, sem)
        g1 = pltpu.async_copy(nf_hbm.at[idx_d], hbuf_d, sem)
        for g in range(CHUNK // 16):
            rs = idx_s[pl.ds(g * 16, 16)]
            rd = idx_d[pl.ds(g * 16, 16)]
            dx = plsc.load_gather(ctx, [rs]) - plsc.load_gather(ctx, [rd])
            dy = plsc.load_gather(cty, [rs]) - plsc.load_gather(cty, [rd])
            dz = plsc.load_gather(ctz, [rs]) - plsc.load_gather(ctz, [rd])
            rad = dx * dx + dy * dy + dz * dz
            cbuf[0, pl.ds(g * 16, 16)] = dx
            cbuf[1, pl.ds(g * 16, 16)] = dy
            cbuf[2, pl.ds(g * 16, 16)] = dz
            cbuf[3, pl.ds(g * 16, 16)] = rad
        w2 = pltpu.async_copy(cbuf, xd_hbm.at[:, pl.ds(base, CHUNK)], sem)
        g0.wait()
        g1.wait()
        w0 = pltpu.async_copy(hbuf_s, hs_hbm.at[pl.ds(base, CHUNK)], sem)
        w1 = pltpu.async_copy(hbuf_d, hd_hbm.at[pl.ds(base, CHUNK)], sem)
        w0.wait(); w1.wait(); w2.wait()
        return carry

    lax.fori_loop(0, nchunks, body, 0)


def _zero_vmem_2d(ref):
    rows, cols = ref.shape
    z = jnp.zeros((16,), F32)

    def rb(r, carry):
        def cb(j, carry2):
            ref[r, pl.ds(j * 16, 16)] = z
            return carry2
        return lax.fori_loop(0, cols // 16, cb, carry)

    lax.fori_loop(0, rows, rb, 0)


def _sc_scatter_body(mh_hbm, mx_hbm, dst_hbm,
                     hn_hbm, xa_hbm,
                     acc, idxb, mbuf, sem):
    c = lax.axis_index("c")
    s = lax.axis_index("s")
    nacc = acc.shape[0]
    ep = mh_hbm.shape[1]
    ept = ep // NSUB
    nchunks = ept // CHUNK
    rows_pt = nacc // NSUB
    nzc = rows_pt // CHUNK

    _zero_vmem_2d(mbuf)

    def zc(k, carry):
        r0 = s * rows_pt + k * CHUNK
        pltpu.sync_copy(mbuf, acc.at[pl.ds(r0, CHUNK)])
        return carry

    lax.fori_loop(0, nzc, zc, 0)
    plsc.subcore_barrier()

    def body(k, carry):
        base = s * ept + k * CHUNK
        pltpu.sync_copy(dst_hbm.at[pl.ds(base, CHUNK)], idxb)
        pltpu.sync_copy(mh_hbm.at[c, pl.ds(base, CHUNK)], mbuf)
        pltpu.sync_copy(mbuf, acc.at[idxb], add=True)
        return carry

    lax.fori_loop(0, nchunks, body, 0)
    plsc.subcore_barrier()

    def co(k, carry):
        r0 = s * rows_pt + k * CHUNK
        pltpu.sync_copy(acc.at[pl.ds(r0, CHUNK)], mbuf)
        pltpu.sync_copy(mbuf, hn_hbm.at[c, pl.ds(r0, CHUNK)])
        return carry

    lax.fori_loop(0, nzc, co, 0)
    plsc.subcore_barrier()

    _zero_vmem_2d(mbuf)
    lax.fori_loop(0, nzc, zc, 0)
    plsc.subcore_barrier()

    ept2 = ep // (2 * NSUB)
    nchunks2 = ept2 // CHUNK

    def body2(k, carry):
        base = c * (ep // 2) + s * ept2 + k * CHUNK
        pltpu.sync_copy(dst_hbm.at[pl.ds(base, CHUNK)], idxb)
        pltpu.sync_copy(mx_hbm.at[pl.ds(base, CHUNK)], mbuf)
        pltpu.sync_copy(mbuf, acc.at[idxb], add=True)
        return carry

    lax.fori_loop(0, nchunks2, body2, 0)
    plsc.subcore_barrier()

    def co2(k, carry):
        r0 = s * rows_pt + k * CHUNK
        pltpu.sync_copy(acc.at[pl.ds(r0, CHUNK)], mbuf)
        pltpu.sync_copy(mbuf, xa_hbm.at[c, pl.ds(r0, CHUNK)])
        return carry

    lax.fori_loop(0, nzc, co2, 0)


def _tc_edge_body(hs_ref, hd_ref, xd_ref, ef_ref,
                  wa_ref, wb_ref, wr_ref, wef_ref, be1_ref,
                  we2_ref, be2_ref, wc1_ref, bc1_ref, wc2_ref,
                  mh_ref, mx_ref):
    v = xd_ref[...]
    radial = v[:, 3:4]
    xdn = v / (jnp.sqrt(radial) + 1e-30)
    t1 = (jnp.dot(hs_ref[...], wa_ref[...], preferred_element_type=F32)
          + jnp.dot(hd_ref[...], wb_ref[...], preferred_element_type=F32)
          + radial * wr_ref[...]
          + jnp.dot(ef_ref[...], wef_ref[...], preferred_element_type=F32)
          + be1_ref[...])
    a1 = _silu(t1)
    t2 = jnp.dot(a1, we2_ref[...], preferred_element_type=F32) + be2_ref[...]
    mh = _silu(t2)
    t3 = jnp.dot(mh, wc1_ref[...], preferred_element_type=F32) + bc1_ref[...]
    a3 = _silu(t3)
    csc = jnp.sum(a3 * wc2_ref[...], axis=1, keepdims=True)
    sca = jnp.tanh(csc) * 10.0
    mx = sca * xdn
    lane = lax.broadcasted_iota(jnp.int32, mx.shape, 1)
    mx = jnp.where(lane == 3, 1.0, mx)
    mh_ref[0] = mh[:, :128]
    mh_ref[1] = mh[:, 128:]
    mx_ref[...] = jnp.concatenate(
        [mx, jnp.zeros((mx.shape[0], 128 - XW), F32)], axis=1)


def _tc_node_body(nf_ref, cp_ref, h0_ref, h1_ref, xa0_ref, xa1_ref,
                  wn1a_ref, wn1b_ref, wn1c_ref, bn1_ref, wn2_ref, bn2_ref,
                  h_ref, x_ref):
    t = (jnp.dot(nf_ref[...], wn1a_ref[...], preferred_element_type=F32)
         + jnp.dot(h0_ref[...], wn1b_ref[...], preferred_element_type=F32)
         + jnp.dot(h1_ref[...], wn1c_ref[...], preferred_element_type=F32)
         + bn1_ref[...])
    a = _silu(t)
    h_ref[...] = jnp.dot(a, wn2_ref[...], preferred_element_type=F32) + bn2_ref[...]
    xa = xa0_ref[...] + xa1_ref[...]
    deg = jnp.maximum(xa[:, 3:4], 1.0)
    x_ref[...] = cp_ref[...] + xa / deg


def kernel(node_feat, coord_feat, edge_feat, We1, be1, We2, be2,
           Wn1, bn1, Wn2, bn2, Wc1, bc1, Wc2, edge_index):
    N, d_in = node_feat.shape
    E = edge_index.shape[1]
    d_h = We2.shape[0]
    d_e = edge_feat.shape[1]
    d_out = Wn2.shape[1]

    grain = NW * CHUNK
    EP = ((E + grain - 1) // grain) * grain
    NACC = ((N + 1 + NSUB * CHUNK - 1) // (NSUB * CHUNK)) * (NSUB * CHUNK)
    BE = 2048
    n_eblk = EP // BE
    BN = 1000
    n_nblk = N // BN

    src = edge_index[0]
    dst = edge_index[1]
    pad = EP - E
    src_p = jnp.concatenate([src, jnp.zeros((pad,), jnp.int32)])
    dst_p = jnp.concatenate([dst, jnp.full((pad,), N, jnp.int32)])
    nf_t = jnp.zeros((NACC, d_in), F32).at[:N].set(node_feat)
    cx = jnp.zeros((NACC,), F32).at[:N].set(coord_feat[:, 0])
    cy = jnp.zeros((NACC,), F32).at[:N].set(coord_feat[:, 1])
    cz = jnp.zeros((NACC,), F32).at[:N].set(coord_feat[:, 2])
    ef_p = jnp.zeros((EP, d_e), F32).at[:E].set(edge_feat)

    mesh = plsc.VectorSubcoreMesh(core_axis_name="c", subcore_axis_name="s")

    gather_fn = pl.kernel(
        _sc_gather_body,
        out_type=(
            jax.ShapeDtypeStruct((EP, d_in), F32),
            jax.ShapeDtypeStruct((EP, d_in), F32),
            jax.ShapeDtypeStruct((XW, EP), F32),
        ),
        mesh=mesh,
        scratch_types=[
            pltpu.VMEM((CHUNK,), jnp.int32),
            pltpu.VMEM((CHUNK,), jnp.int32),
            pltpu.VMEM((CHUNK, d_in), F32),
            pltpu.VMEM((CHUNK, d_in), F32),
            pltpu.VMEM((XW, CHUNK), F32),
            pltpu.VMEM((NACC,), F32),
            pltpu.VMEM((NACC,), F32),
            pltpu.VMEM((NACC,), F32),
            pltpu.SemaphoreType.DMA,
        ],
        compiler_params=pltpu.CompilerParams(needs_layout_passes=False),
    )
    if _DEBUG_XLA_GATHER:
        hs = jnp.take(nf_t, src_p, axis=0)
        hd = jnp.take(nf_t, dst_p, axis=0)
        dxyz = (jnp.take(coord_feat, jnp.minimum(src_p, N - 1), axis=0)
                - jnp.take(coord_feat, jnp.minimum(dst_p, N - 1), axis=0))
        dxyz = jnp.where((src_p < N)[:, None] & (dst_p < N)[:, None], dxyz, 0.0)
        rad = jnp.sum(dxyz * dxyz, axis=1, keepdims=True)
        xdT = jnp.zeros((EP, XW), F32).at[:, :3].set(dxyz).at[:, 3:4].set(rad)
    else:
        hs, hd, xd = gather_fn(nf_t, cx, cy, cz, src_p, dst_p)
        xdT = xd.T

    Wa = We1[:d_in]
    Wb = We1[d_in:2 * d_in]
    wr = We1[2 * d_in:2 * d_in + 1]
    Wef = We1[2 * d_in + 1:]
    be1r = be1.reshape(1, d_h)
    be2r = be2.reshape(1, d_h)
    bc1r = bc1.reshape(1, d_h)
    wc2r = Wc2.reshape(1, d_h)

    eb = lambda i: (i, 0)
    w2 = lambda i: (0, 0)

    mh, mx = pl.pallas_call(
        _tc_edge_body,
        grid=(n_eblk,),
        in_specs=[
            pl.BlockSpec((BE, d_in), eb),
            pl.BlockSpec((BE, d_in), eb),
            pl.BlockSpec((BE, XW), eb),
            pl.BlockSpec((BE, d_e), eb),
            pl.BlockSpec((d_in, d_h), w2),
            pl.BlockSpec((d_in, d_h), w2),
            pl.BlockSpec((1, d_h), w2),
            pl.BlockSpec((d_e, d_h), w2),
            pl.BlockSpec((1, d_h), w2),
            pl.BlockSpec((d_h, d_h), w2),
            pl.BlockSpec((1, d_h), w2),
            pl.BlockSpec((d_h, d_h), w2),
            pl.BlockSpec((1, d_h), w2),
            pl.BlockSpec((1, d_h), w2),
        ],
        out_specs=[
            pl.BlockSpec((2, BE, 128), lambda i: (0, i, 0)),
            pl.BlockSpec((BE, 128), eb),
        ],
        out_shape=[
            jax.ShapeDtypeStruct((2, EP, 128), F32),
            jax.ShapeDtypeStruct((EP, 128), F32),
        ],
        compiler_params=pltpu.CompilerParams(
            dimension_semantics=("parallel",)),
    )(hs, hd, xdT, ef_p, Wa, Wb, wr, Wef, be1r,
      We2, be2r, Wc1, bc1r, wc2r)

    scatter_fn = pl.kernel(
        _sc_scatter_body,
        out_type=(
            jax.ShapeDtypeStruct((2, NACC, 128), F32),
            jax.ShapeDtypeStruct((2, NACC, 128), F32),
        ),
        mesh=mesh,
        scratch_types=[
            pltpu.VMEM_SHARED((NACC, 128), F32),
            pltpu.VMEM((CHUNK,), jnp.int32),
            pltpu.VMEM((CHUNK, 128), F32),
            pltpu.SemaphoreType.DMA,
        ],
    )
    hn, xa = scatter_fn(mh, mx, dst_p)
    if _DEBUG_XLA_SCATTER:
        hn0 = jax.ops.segment_sum(mh[0, :E], dst, num_segments=NACC)
        hn1 = jax.ops.segment_sum(mh[1, :E], dst, num_segments=NACC)
        hn = jnp.stack([hn0, hn1])
        xs = jax.ops.segment_sum(mx[:E], dst, num_segments=NACC)
        xa = jnp.stack([xs, jnp.zeros_like(xs)])
    elif _DEBUG_XLA_XA:
        xs = jax.ops.segment_sum(mx[:E], dst, num_segments=NACC)
        xa = jnp.stack([xs, jnp.zeros_like(xs)])

    Wn1a = Wn1[:d_in]
    Wn1b = Wn1[d_in:d_in + 128]
    Wn1c = Wn1[d_in + 128:]
    bn1r = bn1.reshape(1, d_h)
    bn2r = bn2.reshape(1, d_out)
    cp8 = jnp.zeros((N, XW), F32).at[:, :3].set(coord_feat)

    h, xp = pl.pallas_call(
        _tc_node_body,
        grid=(n_nblk,),
        in_specs=[
            pl.BlockSpec((BN, d_in), eb),
            pl.BlockSpec((BN, XW), eb),
            pl.BlockSpec((BN, 128), eb),
            pl.BlockSpec((BN, 128), eb),
            pl.BlockSpec((BN, XW), eb),
            pl.BlockSpec((BN, XW), eb),
            pl.BlockSpec((d_in, d_h), w2),
            pl.BlockSpec((128, d_h), w2),
            pl.BlockSpec((128, d_h), w2),
            pl.BlockSpec((1, d_h), w2),
            pl.BlockSpec((d_h, d_out), w2),
            pl.BlockSpec((1, d_out), w2),
        ],
        out_specs=[
            pl.BlockSpec((BN, d_out), eb),
            pl.BlockSpec((BN, XW), eb),
        ],
        out_shape=[
            jax.ShapeDtypeStruct((N, d_out), F32),
            jax.ShapeDtypeStruct((N, XW), F32),
        ],
        compiler_params=pltpu.CompilerParams(
            dimension_semantics=("parallel",)),
    )(node_feat, cp8, hn[0, :N], hn[1, :N], xa[0, :N, :XW], xa[1, :N, :XW],
      Wn1a, Wn1b, Wn1c, bn1r, Wn2, bn2r)

    x = xp[:, :3]
    return (h, x)


# speedup vs baseline: 3.7350x; 1.1493x over previous
import jax
import jax.numpy as jnp
from jax import lax
from jax.experimental import pallas as pl
from jax.experimental.pallas import tpu as pltpu
from jax.experimental.pallas import tpu_sc as plsc

F32 = jnp.float32
CHUNK = 128
NCORES = 2
NSUB = 16
NW = NCORES * NSUB
XW = 16


def _silu(x):
    return x * jax.nn.sigmoid(x)


def _sc_gather_body(nf_hbm, cx_hbm, cy_hbm, cz_hbm, src_hbm, dst_hbm,
                    hs_hbm, hd_hbm, xd_hbm,
                    is0, id0, is1, id1, hb_s0, hb_d0, hb_s1, hb_d1,
                    cb0, cb1, ctx, cty, ctz, sem_a, sem_b, sem_w):
    c = lax.axis_index("c")
    s = lax.axis_index("s")
    wid = c * NSUB + s
    ep = hs_hbm.shape[0]
    epw = ep // NW
    nchunks = epw // CHUNK
    base0 = wid * epw

    pltpu.sync_copy(cx_hbm, ctx)
    pltpu.sync_copy(cy_hbm, cty)
    pltpu.sync_copy(cz_hbm, ctz)
    zero16 = jnp.zeros((16,), F32)
    for cb in (cb0, cb1):
        for r in range(4, XW):
            for g in range(CHUNK // 16):
                cb[r, pl.ds(g * 16, 16)] = zero16

    def coords(idx_s, idx_d, cbuf):
        for g in range(CHUNK // 16):
            rs = idx_s[pl.ds(g * 16, 16)]
            rd = idx_d[pl.ds(g * 16, 16)]
            dx = plsc.load_gather(ctx, [rs]) - plsc.load_gather(ctx, [rd])
            dy = plsc.load_gather(cty, [rs]) - plsc.load_gather(cty, [rd])
            dz = plsc.load_gather(ctz, [rs]) - plsc.load_gather(ctz, [rd])
            rad = dx * dx + dy * dy + dz * dz
            cbuf[0, pl.ds(g * 16, 16)] = dx
            cbuf[1, pl.ds(g * 16, 16)] = dy
            cbuf[2, pl.ds(g * 16, 16)] = dz
            cbuf[3, pl.ds(g * 16, 16)] = rad

    def body(p, carry):
        b0 = base0 + (2 * p) * CHUNK
        b1 = b0 + CHUNK
        pltpu.sync_copy(src_hbm.at[pl.ds(b0, CHUNK)], is0)
        pltpu.sync_copy(dst_hbm.at[pl.ds(b0, CHUNK)], id0)
        g0 = pltpu.async_copy(nf_hbm.at[is0], hb_s0, sem_a)
        g1 = pltpu.async_copy(nf_hbm.at[id0], hb_d0, sem_a)
        pltpu.sync_copy(src_hbm.at[pl.ds(b1, CHUNK)], is1)
        pltpu.sync_copy(dst_hbm.at[pl.ds(b1, CHUNK)], id1)
        g2 = pltpu.async_copy(nf_hbm.at[is1], hb_s1, sem_b)
        g3 = pltpu.async_copy(nf_hbm.at[id1], hb_d1, sem_b)
        coords(is0, id0, cb0)
        w0 = pltpu.async_copy(cb0, xd_hbm.at[:, pl.ds(b0, CHUNK)], sem_w)
        coords(is1, id1, cb1)
        w1 = pltpu.async_copy(cb1, xd_hbm.at[:, pl.ds(b1, CHUNK)], sem_w)
        g0.wait()
        g1.wait()
        w2 = pltpu.async_copy(hb_s0, hs_hbm.at[pl.ds(b0, CHUNK)], sem_w)
        w3 = pltpu.async_copy(hb_d0, hd_hbm.at[pl.ds(b0, CHUNK)], sem_w)
        g2.wait()
        g3.wait()
        w4 = pltpu.async_copy(hb_s1, hs_hbm.at[pl.ds(b1, CHUNK)], sem_w)
        w5 = pltpu.async_copy(hb_d1, hd_hbm.at[pl.ds(b1, CHUNK)], sem_w)
        w0.wait(); w1.wait(); w2.wait(); w3.wait(); w4.wait(); w5.wait()
        return carry

    lax.fori_loop(0, nchunks // 2, body, 0)

    if nchunks % 2:
        base = base0 + (nchunks - 1) * CHUNK
        pltpu.sync_copy(src_hbm.at[pl.ds(base, CHUNK)], is0)
        pltpu.sync_copy(dst_hbm.at[pl.ds(base, CHUNK)], id0)
        g0 = pltpu.async_copy(nf_hbm.at[is0], hb_s0, sem_a)
        g1 = pltpu.async_copy(nf_hbm.at[id0], hb_d0, sem_a)
        coords(is0, id0, cb0)
        w0 = pltpu.async_copy(cb0, xd_hbm.at[:, pl.ds(base, CHUNK)], sem_w)
        g0.wait()
        g1.wait()
        w1 = pltpu.async_copy(hb_s0, hs_hbm.at[pl.ds(base, CHUNK)], sem_w)
        w2 = pltpu.async_copy(hb_d0, hd_hbm.at[pl.ds(base, CHUNK)], sem_w)
        w0.wait(); w1.wait(); w2.wait()


def _zero_vmem_2d(ref):
    rows, cols = ref.shape
    z = jnp.zeros((16,), F32)

    def rb(r, carry):
        def cb(j, carry2):
            ref[r, pl.ds(j * 16, 16)] = z
            return carry2
        return lax.fori_loop(0, cols // 16, cb, carry)

    lax.fori_loop(0, rows, rb, 0)


def _sc_scatter_body(mh_hbm, mx_hbm, dst_hbm,
                     hn_hbm, xa_hbm,
                     acc, ib0, ib1, mb0, mb1, sem_a, sem_b):
    c = lax.axis_index("c")
    s = lax.axis_index("s")
    nacc = acc.shape[0]
    ep = mh_hbm.shape[1]
    ept = ep // NSUB
    nchunks = ept // CHUNK
    rows_pt = nacc // NSUB
    nzc = rows_pt // CHUNK

    _zero_vmem_2d(mb0)

    def zc(k, carry):
        r0 = s * rows_pt + k * CHUNK
        pltpu.sync_copy(mb0, acc.at[pl.ds(r0, CHUNK)])
        return carry

    lax.fori_loop(0, nzc, zc, 0)
    plsc.subcore_barrier()

    def body(p, carry):
        b0 = s * ept + (2 * p) * CHUNK
        b1 = b0 + CHUNK
        a0 = pltpu.async_copy(dst_hbm.at[pl.ds(b0, CHUNK)], ib0, sem_a)
        a1 = pltpu.async_copy(mh_hbm.at[c, pl.ds(b0, CHUNK)], mb0, sem_a)
        a2 = pltpu.async_copy(dst_hbm.at[pl.ds(b1, CHUNK)], ib1, sem_b)
        a3 = pltpu.async_copy(mh_hbm.at[c, pl.ds(b1, CHUNK)], mb1, sem_b)
        a0.wait(); a1.wait()
        pltpu.sync_copy(mb0, acc.at[ib0], add=True)
        a2.wait(); a3.wait()
        pltpu.sync_copy(mb1, acc.at[ib1], add=True)
        return carry

    lax.fori_loop(0, nchunks // 2, body, 0)

    if nchunks % 2:
        base = s * ept + (nchunks - 1) * CHUNK
        pltpu.sync_copy(dst_hbm.at[pl.ds(base, CHUNK)], ib0)
        pltpu.sync_copy(mh_hbm.at[c, pl.ds(base, CHUNK)], mb0)
        pltpu.sync_copy(mb0, acc.at[ib0], add=True)
    plsc.subcore_barrier()

    def co(k, carry):
        r0 = s * rows_pt + k * CHUNK
        pltpu.sync_copy(acc.at[pl.ds(r0, CHUNK)], hn_hbm.at[c, pl.ds(r0, CHUNK)])
        return carry

    lax.fori_loop(0, nzc, co, 0)
    plsc.subcore_barrier()

    _zero_vmem_2d(mb0)
    lax.fori_loop(0, nzc, zc, 0)
    plsc.subcore_barrier()

    ept2 = ep // (2 * NSUB)
    nchunks2 = ept2 // CHUNK

    def body2(p, carry):
        b0 = c * (ep // 2) + s * ept2 + (2 * p) * CHUNK
        b1 = b0 + CHUNK
        a0 = pltpu.async_copy(dst_hbm.at[pl.ds(b0, CHUNK)], ib0, sem_a)
        a1 = pltpu.async_copy(mx_hbm.at[pl.ds(b0, CHUNK)], mb0, sem_a)
        a2 = pltpu.async_copy(dst_hbm.at[pl.ds(b1, CHUNK)], ib1, sem_b)
        a3 = pltpu.async_copy(mx_hbm.at[pl.ds(b1, CHUNK)], mb1, sem_b)
        a0.wait(); a1.wait()
        pltpu.sync_copy(mb0, acc.at[ib0], add=True)
        a2.wait(); a3.wait()
        pltpu.sync_copy(mb1, acc.at[ib1], add=True)
        return carry

    lax.fori_loop(0, nchunks2 // 2, body2, 0)

    if nchunks2 % 2:
        base = c * (ep // 2) + s * ept2 + (nchunks2 - 1) * CHUNK
        pltpu.sync_copy(dst_hbm.at[pl.ds(base, CHUNK)], ib0)
        pltpu.sync_copy(mx_hbm.at[pl.ds(base, CHUNK)], mb0)
        pltpu.sync_copy(mb0, acc.at[ib0], add=True)
    plsc.subcore_barrier()

    def co2(k, carry):
        r0 = s * rows_pt + k * CHUNK
        pltpu.sync_copy(acc.at[pl.ds(r0, CHUNK)], xa_hbm.at[c, pl.ds(r0, CHUNK)])
        return carry

    lax.fori_loop(0, nzc, co2, 0)


def _tc_edge_body(hs_ref, hd_ref, xd_ref, ef_ref,
                  wa_ref, wb_ref, wr_ref, wef_ref, be1_ref,
                  we2_ref, be2_ref, wc1_ref, bc1_ref, wc2_ref,
                  mh_ref, mx_ref):
    v = xd_ref[...]
    radial = v[:, 3:4]
    xdn = v / (jnp.sqrt(radial) + 1e-30)
    t1 = (jnp.dot(hs_ref[...], wa_ref[...], preferred_element_type=F32)
          + jnp.dot(hd_ref[...], wb_ref[...], preferred_element_type=F32)
          + radial * wr_ref[...]
          + jnp.dot(ef_ref[...], wef_ref[...], preferred_element_type=F32)
          + be1_ref[...])
    a1 = _silu(t1)
    t2 = jnp.dot(a1, we2_ref[...], preferred_element_type=F32) + be2_ref[...]
    mh = _silu(t2)
    t3 = jnp.dot(mh, wc1_ref[...], preferred_element_type=F32) + bc1_ref[...]
    a3 = _silu(t3)
    csc = jnp.sum(a3 * wc2_ref[...], axis=1, keepdims=True)
    sca = jnp.tanh(csc) * 10.0
    mx = sca * xdn
    lane = lax.broadcasted_iota(jnp.int32, mx.shape, 1)
    mx = jnp.where(lane == 3, 1.0, mx)
    mh_ref[0] = mh[:, :128]
    mh_ref[1] = mh[:, 128:]
    mx_ref[...] = jnp.concatenate(
        [mx, jnp.zeros((mx.shape[0], 128 - XW), F32)], axis=1)


def _tc_node_body(nf_ref, cp_ref, h0_ref, h1_ref, xa0_ref, xa1_ref,
                  wn1a_ref, wn1b_ref, wn1c_ref, bn1_ref, wn2_ref, bn2_ref,
                  h_ref, x_ref):
    t = (jnp.dot(nf_ref[...], wn1a_ref[...], preferred_element_type=F32)
         + jnp.dot(h0_ref[...], wn1b_ref[...], preferred_element_type=F32)
         + jnp.dot(h1_ref[...], wn1c_ref[...], preferred_element_type=F32)
         + bn1_ref[...])
    a = _silu(t)
    h_ref[...] = jnp.dot(a, wn2_ref[...], preferred_element_type=F32) + bn2_ref[...]
    xa = xa0_ref[...] + xa1_ref[...]
    deg = jnp.maximum(xa[:, 3:4], 1.0)
    x_ref[...] = cp_ref[...] + xa / deg


def kernel(node_feat, coord_feat, edge_feat, We1, be1, We2, be2,
           Wn1, bn1, Wn2, bn2, Wc1, bc1, Wc2, edge_index):
    N, d_in = node_feat.shape
    E = edge_index.shape[1]
    d_h = We2.shape[0]
    d_e = edge_feat.shape[1]
    d_out = Wn2.shape[1]

    grain = NW * CHUNK
    EP = ((E + grain - 1) // grain) * grain
    NACC = ((N + 1 + NSUB * CHUNK - 1) // (NSUB * CHUNK)) * (NSUB * CHUNK)
    BE = 2048
    n_eblk = EP // BE
    BN = 1000
    n_nblk = N // BN

    src = edge_index[0]
    dst = edge_index[1]
    pad = EP - E
    src_p = jnp.concatenate([src, jnp.zeros((pad,), jnp.int32)])
    dst_p = jnp.concatenate([dst, jnp.full((pad,), N, jnp.int32)])
    nf_t = jnp.zeros((NACC, d_in), F32).at[:N].set(node_feat)
    cx = jnp.zeros((NACC,), F32).at[:N].set(coord_feat[:, 0])
    cy = jnp.zeros((NACC,), F32).at[:N].set(coord_feat[:, 1])
    cz = jnp.zeros((NACC,), F32).at[:N].set(coord_feat[:, 2])
    ef_p = jnp.zeros((EP, d_e), F32).at[:E].set(edge_feat)

    mesh = plsc.VectorSubcoreMesh(core_axis_name="c", subcore_axis_name="s")

    gather_fn = pl.kernel(
        _sc_gather_body,
        out_type=(
            jax.ShapeDtypeStruct((EP, d_in), F32),
            jax.ShapeDtypeStruct((EP, d_in), F32),
            jax.ShapeDtypeStruct((XW, EP), F32),
        ),
        mesh=mesh,
        scratch_types=[
            pltpu.VMEM((CHUNK,), jnp.int32),
            pltpu.VMEM((CHUNK,), jnp.int32),
            pltpu.VMEM((CHUNK,), jnp.int32),
            pltpu.VMEM((CHUNK,), jnp.int32),
            pltpu.VMEM((CHUNK, d_in), F32),
            pltpu.VMEM((CHUNK, d_in), F32),
            pltpu.VMEM((CHUNK, d_in), F32),
            pltpu.VMEM((CHUNK, d_in), F32),
            pltpu.VMEM((XW, CHUNK), F32),
            pltpu.VMEM((XW, CHUNK), F32),
            pltpu.VMEM((NACC,), F32),
            pltpu.VMEM((NACC,), F32),
            pltpu.VMEM((NACC,), F32),
            pltpu.SemaphoreType.DMA,
            pltpu.SemaphoreType.DMA,
            pltpu.SemaphoreType.DMA,
        ],
        compiler_params=pltpu.CompilerParams(needs_layout_passes=False),
    )
    hs, hd, xd = gather_fn(nf_t, cx, cy, cz, src_p, dst_p)
    xdT = xd.T

    Wa = We1[:d_in]
    Wb = We1[d_in:2 * d_in]
    wr = We1[2 * d_in:2 * d_in + 1]
    Wef = We1[2 * d_in + 1:]
    be1r = be1.reshape(1, d_h)
    be2r = be2.reshape(1, d_h)
    bc1r = bc1.reshape(1, d_h)
    wc2r = Wc2.reshape(1, d_h)

    eb = lambda i: (i, 0)
    w2 = lambda i: (0, 0)

    mh, mx = pl.pallas_call(
        _tc_edge_body,
        grid=(n_eblk,),
        in_specs=[
            pl.BlockSpec((BE, d_in), eb),
            pl.BlockSpec((BE, d_in), eb),
            pl.BlockSpec((BE, XW), eb),
            pl.BlockSpec((BE, d_e), eb),
            pl.BlockSpec((d_in, d_h), w2),
            pl.BlockSpec((d_in, d_h), w2),
            pl.BlockSpec((1, d_h), w2),
            pl.BlockSpec((d_e, d_h), w2),
            pl.BlockSpec((1, d_h), w2),
            pl.BlockSpec((d_h, d_h), w2),
            pl.BlockSpec((1, d_h), w2),
            pl.BlockSpec((d_h, d_h), w2),
            pl.BlockSpec((1, d_h), w2),
            pl.BlockSpec((1, d_h), w2),
        ],
        out_specs=[
            pl.BlockSpec((2, BE, 128), lambda i: (0, i, 0)),
            pl.BlockSpec((BE, 128), eb),
        ],
        out_shape=[
            jax.ShapeDtypeStruct((2, EP, 128), F32),
            jax.ShapeDtypeStruct((EP, 128), F32),
        ],
        compiler_params=pltpu.CompilerParams(
            dimension_semantics=("parallel",)),
    )(hs, hd, xdT, ef_p, Wa, Wb, wr, Wef, be1r,
      We2, be2r, Wc1, bc1r, wc2r)

    scatter_fn = pl.kernel(
        _sc_scatter_body,
        out_type=(
            jax.ShapeDtypeStruct((2, NACC, 128), F32),
            jax.ShapeDtypeStruct((2, NACC, 128), F32),
        ),
        mesh=mesh,
        scratch_types=[
            pltpu.VMEM_SHARED((NACC, 128), F32),
            pltpu.VMEM((CHUNK,), jnp.int32),
            pltpu.VMEM((CHUNK,), jnp.int32),
            pltpu.VMEM((CHUNK, 128), F32),
            pltpu.VMEM((CHUNK, 128), F32),
            pltpu.SemaphoreType.DMA,
            pltpu.SemaphoreType.DMA,
        ],
    )
    hn, xa = scatter_fn(mh, mx, dst_p)

    Wn1a = Wn1[:d_in]
    Wn1b = Wn1[d_in:d_in + 128]
    Wn1c = Wn1[d_in + 128:]
    bn1r = bn1.reshape(1, d_h)
    bn2r = bn2.reshape(1, d_out)
    cp8 = jnp.zeros((N, XW), F32).at[:, :3].set(coord_feat)

    h, xp = pl.pallas_call(
        _tc_node_body,
        grid=(n_nblk,),
        in_specs=[
            pl.BlockSpec((BN, d_in), eb),
            pl.BlockSpec((BN, XW), eb),
            pl.BlockSpec((BN, 128), eb),
            pl.BlockSpec((BN, 128), eb),
            pl.BlockSpec((BN, XW), eb),
            pl.BlockSpec((BN, XW), eb),
            pl.BlockSpec((d_in, d_h), w2),
            pl.BlockSpec((128, d_h), w2),
            pl.BlockSpec((128, d_h), w2),
            pl.BlockSpec((1, d_h), w2),
            pl.BlockSpec((d_h, d_out), w2),
            pl.BlockSpec((1, d_out), w2),
        ],
        out_specs=[
            pl.BlockSpec((BN, d_out), eb),
            pl.BlockSpec((BN, XW), eb),
        ],
        out_shape=[
            jax.ShapeDtypeStruct((N, d_out), F32),
            jax.ShapeDtypeStruct((N, XW), F32),
        ],
        compiler_params=pltpu.CompilerParams(
            dimension_semantics=("parallel",)),
    )(node_feat, cp8, hn[0, :N], hn[1, :N], xa[0, :N, :XW], xa[1, :N, :XW],
      Wn1a, Wn1b, Wn1c, bn1r, Wn2, bn2r)

    x = xp[:, :3]
    return (h, x)
